# baseline (speedup 1.0000x reference)
import functools

import jax
import jax.numpy as jnp
from jax import lax
from jax.experimental import pallas as pl
from jax.experimental.pallas import tpu as pltpu
from jax.experimental.pallas import tpu_sc as plsc

EMBED = 64
LANES = 16
NCORES = 2
NSUB = 16
NWORKERS = NCORES * NSUB
IDX_CHUNK = 128

_PARAMS = pltpu.CompilerParams(needs_layout_passes=False,
                               use_tc_tiling_on_sc=False)


def _gather_body(bpw, nchunk, tid_hbm, table, rows_hbm, idx, rows_v, sem):
    wid = lax.axis_index("s") * NCORES + lax.axis_index("c")
    base = wid * bpw
    for j in range(nchunk):
        pltpu.sync_copy(tid_hbm.at[pl.ds(base + j * IDX_CHUNK, IDX_CHUNK)],
                        idx.at[j])
    copies = []
    for j in range(nchunk):
        copies.append(pltpu.async_copy(
            table.at[idx.at[j]],
            rows_v.at[pl.ds(j * IDX_CHUNK, IDX_CHUNK)], sem))
    for cp in copies:
        cp.wait()
    pltpu.sync_copy(rows_v, rows_hbm.at[pl.ds(base, bpw), :])


def _make_gather(batch, mesh, bpw, nchunk):
    return pl.kernel(
        functools.partial(_gather_body, bpw, nchunk),
        out_type=jax.ShapeDtypeStruct((batch, EMBED), jnp.float32),
        mesh=mesh,
        scratch_types=[
            pltpu.VMEM((nchunk, IDX_CHUNK), jnp.int32),
            pltpu.VMEM((bpw, EMBED), jnp.float32),
            pltpu.SemaphoreType.DMA,
        ],
        compiler_params=_PARAMS,
    )


def kernel(target_ids, context_ids, in_embed, out_embed):
    batch = target_ids.shape[0]
    bpw = batch // NWORKERS
    nchunk = bpw // IDX_CHUNK
    mesh = plsc.VectorSubcoreMesh(core_axis_name="c", subcore_axis_name="s")
    g1 = _make_gather(batch, mesh, bpw, nchunk)
    g2 = _make_gather(batch, mesh, bpw, nchunk)
    rows_in = g1(target_ids.astype(jnp.int32), in_embed)
    rows_out = g2(context_ids.astype(jnp.int32), out_embed)
    return jnp.sum(rows_in * rows_out, axis=1)
